# baseline (speedup 1.0000x reference)
import functools

import jax
import jax.numpy as jnp
from jax import lax
from jax.experimental import pallas as pl
from jax.experimental.pallas import tpu as pltpu
from jax.experimental.pallas import tpu_sc as plsc

N = 10000
E = 320000
D = 128

NC = 2
NS = 16
NW = NC * NS

BLK = 128
SUB = 64
NBLOCK = E // BLK
BPW = NBLOCK // NW
BPW_REM = NBLOCK - BPW * NW
NBUF = 4

ROWS_PER_TILE = 632


def _spmm_body(x_hbm, ei_hbm, adj_hbm, out_hbm,
               rc_v, adj_v, msgs, agg_sh, *sems):
  gsem = sems[0:NBUF]
  ssem = sems[NBUF:2 * NBUF]
  esem = sems[2 * NBUF:3 * NBUF]
  zsem = sems[3 * NBUF]
  c = lax.axis_index("c")
  s = lax.axis_index("s")
  wid = s * NC + c
  r0 = jnp.minimum(s * ROWS_PER_TILE, N - ROWS_PER_TILE)

  base_blk = wid * BPW + jnp.minimum(wid, BPW_REM)
  nblocks = BPW + (wid < BPW_REM).astype(jnp.int32)
  nsub = 2 * nblocks

  def block_copies(blk, slot):
    off = (base_blk + blk) * BLK
    return [
        pltpu.make_async_copy(ei_hbm.at[:, pl.ds(off, BLK)], rc_v.at[slot],
                              esem[slot]),
        pltpu.make_async_copy(adj_hbm.at[pl.ds(off, BLK)], adj_v.at[slot],
                              esem[slot]),
    ]

  def gather_desc(slot, h, m):
    idx = rc_v.at[slot, 1].at[pl.ds(SUB * h, SUB)]
    return pltpu.make_async_copy(x_hbm.at[idx], msgs.at[m], gsem[m])

  def scatter_desc(slot, h, m):
    idx = rc_v.at[slot, 0].at[pl.ds(SUB * h, SUB)]
    return pltpu.make_async_copy(msgs.at[m], agg_sh.at[idx], ssem[m])

  def prep_gather(slot, h, m):
    gather_desc(slot, h, m).start()

  def scale_sub(m, slot, h):
    @plsc.parallel_loop(0, SUB // 16, 1)
    def grp_body(grp):
      a16 = adj_v[slot, pl.ds(SUB * h + 16 * grp, 16)]
      for e in range(16):
        ae = a16[e]
        row = 16 * grp + e
        for g in range(D // 16):
          sl = pl.ds(16 * g, 16)
          msgs[m, row, sl] = msgs[m, row, sl] * ae

  def zero_body(z, carry):
    for j in range(D // 16):
      msgs[NBUF - 1, z, pl.ds(16 * j, 16)] = jnp.zeros((16,), jnp.float32)
    return carry

  lax.fori_loop(0, SUB, zero_body, 0)

  zcopies = [
      pltpu.make_async_copy(msgs.at[NBUF - 1],
                            agg_sh.at[pl.ds(r0 + i * SUB, SUB)], zsem)
      for i in range(9)
  ]
  zcopies.append(pltpu.make_async_copy(
      msgs.at[NBUF - 1].at[pl.ds(0, 56)],
      agg_sh.at[pl.ds(r0 + 9 * SUB, 56)], zsem))
  for d in zcopies:
    d.start()

  for blk in range(3):
    for d in block_copies(jnp.int32(blk), blk):
      d.start()
  for d in block_copies(jnp.int32(0), 0):
    d.wait()
  prep_gather(0, 0, 0)
  prep_gather(0, 1, 1)
  for d in block_copies(jnp.int32(1), 1):
    d.wait()
  prep_gather(1, 0, 2)

  for d in zcopies:
    d.wait()
  plsc.subcore_barrier()

  def step(j, i):
    m = i % NBUF
    h = i % 2
    slot = (i // 2) % NBUF
    gather_desc(slot, h, m).wait()
    scale_sub(m, slot, h)
    pltpu.async_copy(msgs.at[m],
                     agg_sh.at[rc_v.at[slot, 0].at[pl.ds(SUB * h, SUB)]],
                     ssem[m], add=True)

    m3 = (m + 3) % NBUF
    h3 = (i + 3) % 2
    slot3 = ((i + 3) // 2) % NBUF
    i_prev = (i + 7) % (2 * NBUF)
    j3 = j + 3

    @pl.when(jnp.logical_and(j >= 1, j3 < nsub))
    def _wait_prev_scatter():
      scatter_desc((i_prev // 2) % NBUF, i_prev % 2, m3).wait()

    @pl.when(j3 < nsub)
    def _prep():
      blk3 = j3 // 2
      if h3 == 0:
        for d in block_copies(blk3, slot3):
          d.wait()
      prep_gather(slot3, h3, m3)
      if h3 == 1:
        @pl.when(blk3 + 2 < nblocks)
        def _start_block():
          for d in block_copies(blk3 + 2, (slot3 + 2) % NBUF):
            d.start()

  def octo_body(p, carry):
    for i in range(2 * NBUF):
      step(p * 2 * NBUF + i, i)
    return carry

  lax.fori_loop(0, nsub // (2 * NBUF), octo_body, 0)

  tbase = (nsub // (2 * NBUF)) * 2 * NBUF
  for i in range(4):
    step(tbase + i, i)
  for i in range(4, 6):
    jt = tbase + i

    @pl.when(jt < nsub)
    def _tail(jt=jt, i=i):
      step(jt, i)

  for m in range(NBUF):
    scatter_desc(0, 0, m).wait()

  plsc.subcore_barrier()
  pltpu.sync_copy(agg_sh.at[pl.ds(r0, ROWS_PER_TILE)],
                  out_hbm.at[c, pl.ds(r0, ROWS_PER_TILE)])


_spmm = functools.partial(
    pl.kernel,
    out_type=jax.ShapeDtypeStruct((NC, N, D), jnp.float32),
    mesh=plsc.VectorSubcoreMesh(core_axis_name="c", subcore_axis_name="s"),
    scratch_types=[
        pltpu.VMEM((NBUF, 2, BLK), jnp.int32),
        pltpu.VMEM((NBUF, BLK), jnp.float32),
        pltpu.VMEM((NBUF, SUB, D), jnp.float32),
        pltpu.VMEM_SHARED((N, D), jnp.float32),
    ] + [pltpu.SemaphoreType.DMA] * (3 * NBUF + 1),
)(_spmm_body)


def _matmul_relu_body(agg_ref, w_ref, o_ref):
  a = agg_ref[0] + agg_ref[1]
  o_ref[...] = jnp.maximum(
      jnp.dot(a, w_ref[...], preferred_element_type=jnp.float32), 0.0)


BM = 1000


def _matmul_relu(agg, w):
  return pl.pallas_call(
      _matmul_relu_body,
      grid=(N // BM,),
      in_specs=[
          pl.BlockSpec((NC, BM, D), lambda i: (0, i, 0)),
          pl.BlockSpec((D, D), lambda i: (0, 0)),
      ],
      out_specs=pl.BlockSpec((BM, D), lambda i: (i, 0)),
      out_shape=jax.ShapeDtypeStruct((N, D), jnp.float32),
  )(agg, w)


@jax.jit
def kernel(x, edge_index, adj_values, W):
  agg = _spmm(x, edge_index, adj_values)
  return _matmul_relu(agg, W)


# speedup vs baseline: 1.0951x; 1.0951x over previous
import functools

import jax
import jax.numpy as jnp
from jax import lax
from jax.experimental import pallas as pl
from jax.experimental.pallas import tpu as pltpu
from jax.experimental.pallas import tpu_sc as plsc

N = 10000
E = 320000
D = 128

NC = 2
NS = 16
NW = NC * NS

BLK = 128
SUB = 64
NBLOCK = E // BLK
BPW = NBLOCK // NW
BPW_REM = NBLOCK - BPW * NW
NBUF = 4

ROWS_PER_TILE = 632


def _spmm_body(x_hbm, ei_hbm, adj_hbm, out_hbm,
               rc_v, adj_v, msgs, agg_sh, *sems):
  gsem = sems[0:NBUF]
  ssem = sems[NBUF:2 * NBUF]
  esem = sems[2 * NBUF:3 * NBUF]
  zsem = sems[3 * NBUF]
  c = lax.axis_index("c")
  s = lax.axis_index("s")
  wid = s * NC + c
  r0 = jnp.minimum(s * ROWS_PER_TILE, N - ROWS_PER_TILE)

  base_blk = wid * BPW + jnp.minimum(wid, BPW_REM)
  nblocks = BPW + (wid < BPW_REM).astype(jnp.int32)
  nsub = 2 * nblocks

  def block_copies(blk, slot):
    off = (base_blk + blk) * BLK
    return [
        pltpu.make_async_copy(ei_hbm.at[:, pl.ds(off, BLK)], rc_v.at[slot],
                              esem[slot]),
        pltpu.make_async_copy(adj_hbm.at[pl.ds(off, BLK)], adj_v.at[slot],
                              esem[slot]),
    ]

  def gather_desc(slot, h, m):
    idx = rc_v.at[slot, 1].at[pl.ds(SUB * h, SUB)]
    return pltpu.make_async_copy(x_hbm.at[idx], msgs.at[m], gsem[m])

  def scatter_desc(slot, h, m):
    idx = rc_v.at[slot, 0].at[pl.ds(SUB * h, SUB)]
    return pltpu.make_async_copy(msgs.at[m], agg_sh.at[idx], ssem[m])

  def prep_gather(slot, h, m):
    gather_desc(slot, h, m).start()

  def scale_sub(m, slot, h):
    def grp_body(grp, carry):
      a16 = adj_v[slot, pl.ds(SUB * h + 16 * grp, 16)]
      for e in range(16):
        ae = a16[e]
        row = 16 * grp + e
        for g in range(D // 16):
          sl = pl.ds(16 * g, 16)
          msgs[m, row, sl] = msgs[m, row, sl] * ae
      return carry

    lax.fori_loop(0, SUB // 16, grp_body, 0)

  def zero_body(z, carry):
    for j in range(D // 16):
      msgs[NBUF - 1, z, pl.ds(16 * j, 16)] = jnp.zeros((16,), jnp.float32)
    return carry

  lax.fori_loop(0, SUB, zero_body, 0)

  zcopies = [
      pltpu.make_async_copy(msgs.at[NBUF - 1],
                            agg_sh.at[pl.ds(r0 + i * SUB, SUB)], zsem)
      for i in range(9)
  ]
  zcopies.append(pltpu.make_async_copy(
      msgs.at[NBUF - 1].at[pl.ds(0, 56)],
      agg_sh.at[pl.ds(r0 + 9 * SUB, 56)], zsem))
  for d in zcopies:
    d.start()

  for blk in range(3):
    for d in block_copies(jnp.int32(blk), blk):
      d.start()
  for d in block_copies(jnp.int32(0), 0):
    d.wait()
  prep_gather(0, 0, 0)
  prep_gather(0, 1, 1)
  for d in block_copies(jnp.int32(1), 1):
    d.wait()
  prep_gather(1, 0, 2)

  for d in zcopies:
    d.wait()
  plsc.subcore_barrier()

  def step(j, i):
    m = i % NBUF
    h = i % 2
    slot = (i // 2) % NBUF
    gather_desc(slot, h, m).wait()
    scale_sub(m, slot, h)
    pltpu.async_copy(msgs.at[m],
                     agg_sh.at[rc_v.at[slot, 0].at[pl.ds(SUB * h, SUB)]],
                     ssem[m], add=True)

    m3 = (m + 3) % NBUF
    h3 = (i + 3) % 2
    slot3 = ((i + 3) // 2) % NBUF
    i_prev = (i + 7) % (2 * NBUF)
    j3 = j + 3

    @pl.when(jnp.logical_and(j >= 1, j3 < nsub))
    def _wait_prev_scatter():
      scatter_desc((i_prev // 2) % NBUF, i_prev % 2, m3).wait()

    @pl.when(j3 < nsub)
    def _prep():
      blk3 = j3 // 2
      if h3 == 0:
        for d in block_copies(blk3, slot3):
          d.wait()
      prep_gather(slot3, h3, m3)
      if h3 == 1:
        @pl.when(blk3 + 2 < nblocks)
        def _start_block():
          for d in block_copies(blk3 + 2, (slot3 + 2) % NBUF):
            d.start()

  def octo_body(p, carry):
    for i in range(2 * NBUF):
      step(p * 2 * NBUF + i, i)
    return carry

  lax.fori_loop(0, nsub // (2 * NBUF), octo_body, 0)

  tbase = (nsub // (2 * NBUF)) * 2 * NBUF
  for i in range(4):
    step(tbase + i, i)
  for i in range(4, 6):
    jt = tbase + i

    @pl.when(jt < nsub)
    def _tail(jt=jt, i=i):
      step(jt, i)

  for m in range(NBUF):
    scatter_desc(0, 0, m).wait()

  plsc.subcore_barrier()
  pltpu.sync_copy(agg_sh.at[pl.ds(r0, ROWS_PER_TILE)],
                  out_hbm.at[c, pl.ds(r0, ROWS_PER_TILE)])


_spmm = functools.partial(
    pl.kernel,
    out_type=jax.ShapeDtypeStruct((NC, N, D), jnp.float32),
    mesh=plsc.VectorSubcoreMesh(core_axis_name="c", subcore_axis_name="s"),
    scratch_types=[
        pltpu.VMEM((NBUF, 2, BLK), jnp.int32),
        pltpu.VMEM((NBUF, BLK), jnp.float32),
        pltpu.VMEM((NBUF, SUB, D), jnp.float32),
        pltpu.VMEM_SHARED((N, D), jnp.float32),
    ] + [pltpu.SemaphoreType.DMA] * (3 * NBUF + 1),
)(_spmm_body)


def _matmul_relu_body(agg_ref, w_ref, o_ref):
  a = agg_ref[0] + agg_ref[1]
  o_ref[...] = jnp.maximum(
      jnp.dot(a, w_ref[...], preferred_element_type=jnp.float32), 0.0)


BM = 1000


def _matmul_relu(agg, w):
  return pl.pallas_call(
      _matmul_relu_body,
      grid=(N // BM,),
      in_specs=[
          pl.BlockSpec((NC, BM, D), lambda i: (0, i, 0)),
          pl.BlockSpec((D, D), lambda i: (0, 0)),
      ],
      out_specs=pl.BlockSpec((BM, D), lambda i: (i, 0)),
      out_shape=jax.ShapeDtypeStruct((N, D), jnp.float32),
  )(agg, w)


@jax.jit
def kernel(x, edge_index, adj_values, W):
  agg = _spmm(x, edge_index, adj_values)
  return _matmul_relu(agg, W)


# speedup vs baseline: 1.0971x; 1.0018x over previous
import functools

import jax
import jax.numpy as jnp
from jax import lax
from jax.experimental import pallas as pl
from jax.experimental.pallas import tpu as pltpu
from jax.experimental.pallas import tpu_sc as plsc

N = 10000
E = 320000
D = 128

NC = 2
NS = 16
NW = NC * NS

BLK = 128
SUB = 64
NBLOCK = E // BLK
BPW = NBLOCK // NW
BPW_REM = NBLOCK - BPW * NW
NBUF = 4

ROWS_PER_TILE = 632


def _spmm_body(x_hbm, ei_hbm, adj_hbm, out_hbm,
               rc_v, adj_v, msgs, agg_sh, *sems):
  gsem = sems[0:NBUF]
  ssem = sems[NBUF:2 * NBUF]
  esem = sems[2 * NBUF:3 * NBUF]
  zsem = sems[3 * NBUF]
  c = lax.axis_index("c")
  s = lax.axis_index("s")
  wid = s * NC + c
  r0 = jnp.minimum(s * ROWS_PER_TILE, N - ROWS_PER_TILE)

  base_blk = wid * BPW + jnp.minimum(wid, BPW_REM)
  nblocks = BPW + (wid < BPW_REM).astype(jnp.int32)
  nsub = 2 * nblocks

  def block_copies(blk, slot):
    off = (base_blk + blk) * BLK
    return [
        pltpu.make_async_copy(ei_hbm.at[:, pl.ds(off, BLK)], rc_v.at[slot],
                              esem[slot]),
        pltpu.make_async_copy(adj_hbm.at[pl.ds(off, BLK)], adj_v.at[slot],
                              esem[slot]),
    ]

  def gather_desc(slot, h, m):
    idx = rc_v.at[slot, 1].at[pl.ds(SUB * h, SUB)]
    return pltpu.make_async_copy(x_hbm.at[idx], msgs.at[m], gsem[m])

  def scatter_desc(slot, h, m):
    idx = rc_v.at[slot, 0].at[pl.ds(SUB * h, SUB)]
    return pltpu.make_async_copy(msgs.at[m], agg_sh.at[idx], ssem[m])

  def prep_gather(slot, h, m):
    gather_desc(slot, h, m).start()

  def scale_sub(m, slot, h):
    def grp_body(grp, carry):
      a16 = adj_v[slot, pl.ds(SUB * h + 16 * grp, 16)]
      for e in range(16):
        ae = a16[e]
        row = 16 * grp + e
        for g in range(D // 16):
          sl = pl.ds(16 * g, 16)
          msgs[m, row, sl] = msgs[m, row, sl] * ae
      return carry

    lax.fori_loop(0, SUB // 16, grp_body, 0, unroll=2)

  def zero_body(z, carry):
    for j in range(D // 16):
      msgs[NBUF - 1, z, pl.ds(16 * j, 16)] = jnp.zeros((16,), jnp.float32)
    return carry

  lax.fori_loop(0, SUB, zero_body, 0)

  zcopies = [
      pltpu.make_async_copy(msgs.at[NBUF - 1],
                            agg_sh.at[pl.ds(r0 + i * SUB, SUB)], zsem)
      for i in range(9)
  ]
  zcopies.append(pltpu.make_async_copy(
      msgs.at[NBUF - 1].at[pl.ds(0, 56)],
      agg_sh.at[pl.ds(r0 + 9 * SUB, 56)], zsem))
  for d in zcopies:
    d.start()

  for blk in range(3):
    for d in block_copies(jnp.int32(blk), blk):
      d.start()
  for d in block_copies(jnp.int32(0), 0):
    d.wait()
  prep_gather(0, 0, 0)
  prep_gather(0, 1, 1)
  for d in block_copies(jnp.int32(1), 1):
    d.wait()
  prep_gather(1, 0, 2)

  for d in zcopies:
    d.wait()
  plsc.subcore_barrier()

  def step(j, i):
    m = i % NBUF
    h = i % 2
    slot = (i // 2) % NBUF
    gather_desc(slot, h, m).wait()
    scale_sub(m, slot, h)
    pltpu.async_copy(msgs.at[m],
                     agg_sh.at[rc_v.at[slot, 0].at[pl.ds(SUB * h, SUB)]],
                     ssem[m], add=True)

    m3 = (m + 3) % NBUF
    h3 = (i + 3) % 2
    slot3 = ((i + 3) // 2) % NBUF
    i_prev = (i + 7) % (2 * NBUF)
    j3 = j + 3

    @pl.when(jnp.logical_and(j >= 1, j3 < nsub))
    def _wait_prev_scatter():
      scatter_desc((i_prev // 2) % NBUF, i_prev % 2, m3).wait()

    @pl.when(j3 < nsub)
    def _prep():
      blk3 = j3 // 2
      if h3 == 0:
        for d in block_copies(blk3, slot3):
          d.wait()
      prep_gather(slot3, h3, m3)
      if h3 == 1:
        @pl.when(blk3 + 2 < nblocks)
        def _start_block():
          for d in block_copies(blk3 + 2, (slot3 + 2) % NBUF):
            d.start()

  def octo_body(p, carry):
    for i in range(2 * NBUF):
      step(p * 2 * NBUF + i, i)
    return carry

  lax.fori_loop(0, nsub // (2 * NBUF), octo_body, 0)

  tbase = (nsub // (2 * NBUF)) * 2 * NBUF
  for i in range(4):
    step(tbase + i, i)
  for i in range(4, 6):
    jt = tbase + i

    @pl.when(jt < nsub)
    def _tail(jt=jt, i=i):
      step(jt, i)

  for m in range(NBUF):
    scatter_desc(0, 0, m).wait()

  plsc.subcore_barrier()
  pltpu.sync_copy(agg_sh.at[pl.ds(r0, ROWS_PER_TILE)],
                  out_hbm.at[c, pl.ds(r0, ROWS_PER_TILE)])


_spmm = functools.partial(
    pl.kernel,
    out_type=jax.ShapeDtypeStruct((NC, N, D), jnp.float32),
    mesh=plsc.VectorSubcoreMesh(core_axis_name="c", subcore_axis_name="s"),
    scratch_types=[
        pltpu.VMEM((NBUF, 2, BLK), jnp.int32),
        pltpu.VMEM((NBUF, BLK), jnp.float32),
        pltpu.VMEM((NBUF, SUB, D), jnp.float32),
        pltpu.VMEM_SHARED((N, D), jnp.float32),
    ] + [pltpu.SemaphoreType.DMA] * (3 * NBUF + 1),
)(_spmm_body)


def _matmul_relu_body(agg_ref, w_ref, o_ref):
  a = agg_ref[0] + agg_ref[1]
  o_ref[...] = jnp.maximum(
      jnp.dot(a, w_ref[...], preferred_element_type=jnp.float32), 0.0)


BM = 1000


def _matmul_relu(agg, w):
  return pl.pallas_call(
      _matmul_relu_body,
      grid=(N // BM,),
      in_specs=[
          pl.BlockSpec((NC, BM, D), lambda i: (0, i, 0)),
          pl.BlockSpec((D, D), lambda i: (0, 0)),
      ],
      out_specs=pl.BlockSpec((BM, D), lambda i: (i, 0)),
      out_shape=jax.ShapeDtypeStruct((N, D), jnp.float32),
  )(agg, w)


@jax.jit
def kernel(x, edge_index, adj_values, W):
  agg = _spmm(x, edge_index, adj_values)
  return _matmul_relu(agg, W)
